# baseline (speedup 1.0000x reference)
import jax
import jax.numpy as jnp
from jax.experimental import pallas as pl
from jax.experimental.pallas import tpu as pltpu

STYLE = 256
NC = 256
NCOMP = 8
B, H, W = 4, 64, 64
NLAB = 16
EPS = 1e-5


def _fused_kernel(seg_ref, sc_ref, ex_ref, fcwt_ref, fcb_ref, wmid_ref,
                  w2_ref, xt_ref, bnw_ref, bnb_ref, cgb_ref, cbb_ref,
                  out_ref, oh_s, taps_s, st_s, scratch_ref):
    s = pl.program_id(0)

    @pl.when(s < B)
    def _prep():
        i = s
        seg = seg_ref[0]
        mask = (seg != 0).astype(jnp.float32)

        jidx = (jax.lax.broadcasted_iota(jnp.int32, (NCOMP, 1, 1), 0) + 1
                ).astype(jnp.float32)
        lab = jnp.max(mask * jidx, axis=0)

        l_iota = jax.lax.broadcasted_iota(jnp.int32, (1, 1, NLAB), 2
                                          ).astype(jnp.float32)
        oh = (lab[:, :, None] == l_iota).astype(jnp.float32)
        zc = jnp.zeros((H, 1, NLAB), jnp.float32)
        ohp = jnp.concatenate([zc, oh, zc], axis=1)
        zr = jnp.zeros((1, W + 2, NLAB), jnp.float32)
        ohp = jnp.concatenate([zr, ohp, zr], axis=0)
        oh_s[pl.ds(i, 1)] = ohp.astype(jnp.bfloat16)[None]

        sc = sc_ref[0]
        sc_mean = jnp.mean(sc, axis=0, keepdims=True)
        mus = []
        for j in range(NCOMP):
            area = jnp.sum(mask[j])
            code_e = jnp.where(ex_ref[0, 0, j] == 1.0, sc[j:j + 1, :],
                               sc[NCOMP:NCOMP + 1, :])
            code = jnp.where(area > 0.0, code_e, sc_mean)
            mu = jnp.dot(code, fcwt_ref[j], preferred_element_type=jnp.float32)
            mu = jnp.maximum(mu + fcb_ref[j:j + 1, :], 0.0)
            mus.append(mu)
        m16 = jnp.concatenate(
            [jnp.zeros((1, STYLE), jnp.float32)] + mus
            + [jnp.zeros((NLAB - 1 - NCOMP, STYLE), jnp.float32)], axis=0)

        tap3 = []
        for dx in range(3):
            rows = [jnp.dot(m16, wmid_ref[dy * 3 + dx],
                            preferred_element_type=jnp.float32)
                    for dy in range(3)]
            tap3.append(jnp.concatenate(rows, axis=0))
        taps_s[pl.ds(i, 1)] = jnp.stack(tap3, axis=0).astype(jnp.bfloat16)[None]

        xb = xt_ref[pl.ds(i, 1)][0]
        st_s[pl.ds(i, 1), :] = jnp.sum(xb, axis=(0, 1))[None]
        st_s[pl.ds(i + B, 1), :] = jnp.sum(xb * xb, axis=(0, 1))[None]

    @pl.when(s >= B)
    def _conv():
        i = s - B
        n = float(B * H * W)
        mean = jnp.sum(st_s[0:B], axis=0).reshape(1, 1, NC) / n
        var = jnp.sum(st_s[B:2 * B], axis=0).reshape(1, 1, NC) / n - mean * mean
        scale = bnw_ref[0].reshape(1, 1, NC) * jax.lax.rsqrt(var + EPS)
        shift = bnb_ref[0].reshape(1, 1, NC) - mean * scale
        bias = (cgb_ref[0] + cbb_ref[0]).reshape(1, NC)

        taps = taps_s[pl.ds(i, 1)][0]

        for k in range(H // 32):
            base = k * 32
            xin = xt_ref[pl.ds(i, 1), base:base + 32][0]
            coarse_c = xin * scale + shift
            scratch_ref[1:33, 1:W + 1, :] = coarse_c.astype(jnp.bfloat16)
            if k == 0:
                scratch_ref[0:1, 1:W + 1, :] = jnp.zeros((1, W, NC),
                                                         jnp.bfloat16)
            else:
                top = xt_ref[pl.ds(i, 1), base - 1:base][0] * scale + shift
                scratch_ref[0:1, 1:W + 1, :] = top.astype(jnp.bfloat16)
            if k == H // 32 - 1:
                scratch_ref[33:34, 1:W + 1, :] = jnp.zeros((1, W, NC),
                                                           jnp.bfloat16)
            else:
                bot = xt_ref[pl.ds(i, 1), base + 32:base + 33][0] * scale + shift
                scratch_ref[33:34, 1:W + 1, :] = bot.astype(jnp.bfloat16)
            zcol = jnp.zeros((34, 1, NC), jnp.bfloat16)
            scratch_ref[:, 0:1, :] = zcol
            scratch_ref[:, W + 1:W + 2, :] = zcol

            oh = oh_s[pl.ds(i, 1), base:base + 34][0]

            acc = jnp.zeros((32 * W, NC), jnp.float32)
            for dx in range(3):
                lhs = scratch_ref[:, dx:dx + W, :]
                ohl = oh[:, dx:dx + W, :]
                v3 = jnp.concatenate([lhs[0:32], lhs[1:33], lhs[2:34]],
                                     axis=2).reshape(32 * W, 3 * NC)
                acc += jnp.dot(v3, w2_ref[dx],
                               preferred_element_type=jnp.float32)
                u3 = jnp.concatenate([ohl[0:32], ohl[1:33], ohl[2:34]],
                                     axis=2).reshape(32 * W, 3 * NLAB)
                acc += jnp.dot(u3, taps[dx],
                               preferred_element_type=jnp.float32)

            out_ref[0, base:base + 32] = (acc + coarse_c.reshape(32 * W, NC)
                                          + bias).reshape(32, W, NC)


def kernel(x, segmap, style_codes, exist_codes, fc_w, fc_b,
           conv_gamma_w, conv_gamma_b, conv_beta_w, conv_beta_b,
           bn_weight, bn_bias):
    xt = jnp.transpose(x, (0, 2, 3, 1))
    exf = exist_codes.astype(jnp.float32).reshape(B, 1, NCOMP)
    fcwt = jnp.transpose(fc_w, (0, 2, 1))
    wmid = jnp.transpose(conv_gamma_w[:, :NC] + conv_beta_w,
                         (2, 3, 1, 0)).reshape(9, NC, NC)
    w2 = jnp.transpose(conv_gamma_w[:, NC:],
                       (3, 2, 1, 0)).reshape(3, 3 * NC, NC).astype(jnp.bfloat16)

    out_nhwc = pl.pallas_call(
        _fused_kernel,
        grid=(2 * B,),
        in_specs=[
            pl.BlockSpec((1, NCOMP, H, W),
                         lambda s: (jnp.minimum(s, B - 1), 0, 0, 0)),
            pl.BlockSpec((1, NCOMP + 1, STYLE),
                         lambda s: (jnp.minimum(s, B - 1), 0, 0)),
            pl.BlockSpec((1, 1, NCOMP),
                         lambda s: (jnp.minimum(s, B - 1), 0, 0)),
            pl.BlockSpec((NCOMP, STYLE, STYLE), lambda s: (0, 0, 0)),
            pl.BlockSpec((NCOMP, STYLE), lambda s: (0, 0)),
            pl.BlockSpec((9, NC, NC), lambda s: (0, 0, 0)),
            pl.BlockSpec((3, 3 * NC, NC), lambda s: (0, 0, 0)),
            pl.BlockSpec((B, H, W, NC), lambda s: (0, 0, 0, 0)),
            pl.BlockSpec((1, NC), lambda s: (0, 0)),
            pl.BlockSpec((1, NC), lambda s: (0, 0)),
            pl.BlockSpec((1, NC), lambda s: (0, 0)),
            pl.BlockSpec((1, NC), lambda s: (0, 0)),
        ],
        out_specs=pl.BlockSpec((1, H, W, NC),
                               lambda s: (jnp.maximum(s - B, 0), 0, 0, 0)),
        out_shape=jax.ShapeDtypeStruct((B, H, W, NC), jnp.float32),
        scratch_shapes=[
            pltpu.VMEM((B, H + 2, W + 2, NLAB), jnp.bfloat16),
            pltpu.VMEM((B, 3, 3 * NLAB, NC), jnp.bfloat16),
            pltpu.VMEM((2 * B, NC), jnp.float32),
            pltpu.VMEM((34, W + 2, NC), jnp.bfloat16),
        ],
    )(segmap, style_codes, exf, fcwt, fc_b, wmid, w2, xt,
      bn_weight.reshape(1, NC), bn_bias.reshape(1, NC),
      conv_gamma_b.reshape(1, NC), conv_beta_b.reshape(1, NC))

    return jnp.transpose(out_nhwc, (0, 3, 1, 2))


# speedup vs baseline: 1.0295x; 1.0295x over previous
import jax
import jax.numpy as jnp
from jax.experimental import pallas as pl
from jax.experimental.pallas import tpu as pltpu

STYLE = 256
NC = 256
NCOMP = 8
B, H, W = 4, 64, 64
NLAB = 16
EPS = 1e-5


def _fused_kernel(seg_ref, sc_ref, ex_ref, fcwt_ref, fcb_ref, wmid_ref,
                  w2_ref, xt_ref, bnw_ref, bnb_ref, cgb_ref, cbb_ref,
                  out_ref, oh_s, taps_s, st_s, scratch_ref):
    s = pl.program_id(0)

    @pl.when(s < B)
    def _prep():
        i = s
        seg = seg_ref[0]
        mask = (seg != 0).astype(jnp.float32)

        jidx = (jax.lax.broadcasted_iota(jnp.int32, (NCOMP, 1, 1), 0) + 1
                ).astype(jnp.float32)
        lab = jnp.max(mask * jidx, axis=0)

        l_iota = jax.lax.broadcasted_iota(jnp.int32, (1, 1, NLAB), 2
                                          ).astype(jnp.float32)
        oh = (lab[:, :, None] == l_iota).astype(jnp.float32)
        zc = jnp.zeros((H, 1, NLAB), jnp.float32)
        ohp = jnp.concatenate([zc, oh, zc], axis=1)
        zr = jnp.zeros((1, W + 2, NLAB), jnp.float32)
        ohp = jnp.concatenate([zr, ohp, zr], axis=0)
        oh_s[pl.ds(i, 1)] = ohp.astype(jnp.bfloat16)[None]

        sc = sc_ref[0]
        sc_mean = jnp.mean(sc, axis=0, keepdims=True)
        mus = []
        for j in range(NCOMP):
            area = jnp.sum(mask[j])
            code_e = jnp.where(ex_ref[0, 0, j] == 1.0, sc[j:j + 1, :],
                               sc[NCOMP:NCOMP + 1, :])
            code = jnp.where(area > 0.0, code_e, sc_mean)
            mu = jnp.dot(code, fcwt_ref[j], preferred_element_type=jnp.float32)
            mu = jnp.maximum(mu + fcb_ref[j:j + 1, :], 0.0)
            mus.append(mu)
        m16 = jnp.concatenate(
            [jnp.zeros((1, STYLE), jnp.float32)] + mus
            + [jnp.zeros((NLAB - 1 - NCOMP, STYLE), jnp.float32)], axis=0)

        tap3 = []
        for dx in range(3):
            rows = [jnp.dot(m16, wmid_ref[dy * 3 + dx],
                            preferred_element_type=jnp.float32)
                    for dy in range(3)]
            tap3.append(jnp.concatenate(rows, axis=0))
        taps_s[pl.ds(i, 1)] = jnp.stack(tap3, axis=0).astype(jnp.bfloat16)[None]

        xb = xt_ref[pl.ds(i, 1)][0]
        st_s[pl.ds(i, 1), :] = jnp.sum(xb, axis=(0, 1))[None]
        st_s[pl.ds(i + B, 1), :] = jnp.sum(xb * xb, axis=(0, 1))[None]

    @pl.when(s >= B)
    def _conv():
        i = s - B
        n = float(B * H * W)
        mean = jnp.sum(st_s[0:B], axis=0).reshape(1, 1, NC) / n
        var = jnp.sum(st_s[B:2 * B], axis=0).reshape(1, 1, NC) / n - mean * mean
        scale = bnw_ref[0].reshape(1, 1, NC) * jax.lax.rsqrt(var + EPS)
        shift = bnb_ref[0].reshape(1, 1, NC) - mean * scale
        bias = (cgb_ref[0] + cbb_ref[0]).reshape(1, NC)

        taps = taps_s[pl.ds(i, 1)][0]

        for k in range(H // 16):
            base = k * 16
            xin = xt_ref[pl.ds(i, 1), base:base + 16][0]
            coarse_c = xin * scale + shift
            scratch_ref[1:17, 1:W + 1, :] = coarse_c.astype(jnp.bfloat16)
            if k == 0:
                scratch_ref[0:1, 1:W + 1, :] = jnp.zeros((1, W, NC),
                                                         jnp.bfloat16)
            else:
                top = xt_ref[pl.ds(i, 1), base - 1:base][0] * scale + shift
                scratch_ref[0:1, 1:W + 1, :] = top.astype(jnp.bfloat16)
            if k == H // 16 - 1:
                scratch_ref[17:18, 1:W + 1, :] = jnp.zeros((1, W, NC),
                                                           jnp.bfloat16)
            else:
                bot = xt_ref[pl.ds(i, 1), base + 16:base + 17][0] * scale + shift
                scratch_ref[17:18, 1:W + 1, :] = bot.astype(jnp.bfloat16)
            zcol = jnp.zeros((18, 1, NC), jnp.bfloat16)
            scratch_ref[:, 0:1, :] = zcol
            scratch_ref[:, W + 1:W + 2, :] = zcol

            oh = oh_s[pl.ds(i, 1), base:base + 18][0]

            acc = jnp.zeros((16 * W, NC), jnp.float32)
            for dx in range(3):
                lhs = scratch_ref[:, dx:dx + W, :]
                ohl = oh[:, dx:dx + W, :]
                v3 = jnp.concatenate([lhs[0:16], lhs[1:17], lhs[2:18]],
                                     axis=2).reshape(16 * W, 3 * NC)
                acc += jnp.dot(v3, w2_ref[dx],
                               preferred_element_type=jnp.float32)
                u3 = jnp.concatenate([ohl[0:16], ohl[1:17], ohl[2:18]],
                                     axis=2).reshape(16 * W, 3 * NLAB)
                acc += jnp.dot(u3, taps[dx],
                               preferred_element_type=jnp.float32)

            out_ref[0, base:base + 16] = (acc + coarse_c.reshape(16 * W, NC)
                                          + bias).reshape(16, W, NC)


def kernel(x, segmap, style_codes, exist_codes, fc_w, fc_b,
           conv_gamma_w, conv_gamma_b, conv_beta_w, conv_beta_b,
           bn_weight, bn_bias):
    xt = jnp.transpose(x, (0, 2, 3, 1))
    exf = exist_codes.astype(jnp.float32).reshape(B, 1, NCOMP)
    fcwt = jnp.transpose(fc_w, (0, 2, 1))
    wmid = jnp.transpose(conv_gamma_w[:, :NC] + conv_beta_w,
                         (2, 3, 1, 0)).reshape(9, NC, NC)
    w2 = jnp.transpose(conv_gamma_w[:, NC:],
                       (3, 2, 1, 0)).reshape(3, 3 * NC, NC).astype(jnp.bfloat16)

    out_nhwc = pl.pallas_call(
        _fused_kernel,
        grid=(2 * B,),
        in_specs=[
            pl.BlockSpec((1, NCOMP, H, W),
                         lambda s: (jnp.minimum(s, B - 1), 0, 0, 0)),
            pl.BlockSpec((1, NCOMP + 1, STYLE),
                         lambda s: (jnp.minimum(s, B - 1), 0, 0)),
            pl.BlockSpec((1, 1, NCOMP),
                         lambda s: (jnp.minimum(s, B - 1), 0, 0)),
            pl.BlockSpec((NCOMP, STYLE, STYLE), lambda s: (0, 0, 0)),
            pl.BlockSpec((NCOMP, STYLE), lambda s: (0, 0)),
            pl.BlockSpec((9, NC, NC), lambda s: (0, 0, 0)),
            pl.BlockSpec((3, 3 * NC, NC), lambda s: (0, 0, 0)),
            pl.BlockSpec((B, H, W, NC), lambda s: (0, 0, 0, 0)),
            pl.BlockSpec((1, NC), lambda s: (0, 0)),
            pl.BlockSpec((1, NC), lambda s: (0, 0)),
            pl.BlockSpec((1, NC), lambda s: (0, 0)),
            pl.BlockSpec((1, NC), lambda s: (0, 0)),
        ],
        out_specs=pl.BlockSpec((1, H, W, NC),
                               lambda s: (jnp.maximum(s - B, 0), 0, 0, 0)),
        out_shape=jax.ShapeDtypeStruct((B, H, W, NC), jnp.float32),
        scratch_shapes=[
            pltpu.VMEM((B, H + 2, W + 2, NLAB), jnp.bfloat16),
            pltpu.VMEM((B, 3, 3 * NLAB, NC), jnp.bfloat16),
            pltpu.VMEM((2 * B, NC), jnp.float32),
            pltpu.VMEM((18, W + 2, NC), jnp.bfloat16),
        ],
    )(segmap, style_codes, exf, fcwt, fc_b, wmid, w2, xt,
      bn_weight.reshape(1, NC), bn_bias.reshape(1, NC),
      conv_gamma_b.reshape(1, NC), conv_beta_b.reshape(1, NC))

    return jnp.transpose(out_nhwc, (0, 3, 1, 2))
